# baseline (speedup 1.0000x reference)
import jax
import jax.numpy as jnp
from jax import lax
from jax.experimental import pallas as pl
from jax.experimental.pallas import tpu as pltpu
from jax.experimental.pallas import tpu_sc as plsc

D = 128
N = 512
NPAD = N + 16
SF = float(D - 1)
INV_SF = 1.0 / SF
NW = 32
SLAB = D // NW
NG = N // 16


def _sread(ref, i):
    return ref[pl.ds(i, 16)][0]


def _sread2(ref, row, i):
    return ref[pl.ds(row * NPAD + i, 16)][0]


def _splat_body(par_h, out_h, par, ipk, ypk, zpk, invr,
                gxa0, gxa1, gxa2, gxa3, fbuf, vol):
    gxa = [gxa0, gxa1, gxa2, gxa3]
    wid = lax.axis_index("s") * 2 + lax.axis_index("c")
    x0 = wid * SLAB
    x0f = x0.astype(jnp.float32)

    pltpu.sync_copy(par_h, par)

    zeros = jnp.zeros((16,), jnp.float32)

    @plsc.parallel_loop(0, SLAB * D, unroll=4)
    def _init(r):
        for c in range(8):
            vol[r, pl.ds(16 * c, 16)] = zeros

    lane = lax.iota(jnp.int32, 16)
    lanef = lane.astype(jnp.float32)

    def _prep(g, cnt):
        sl = pl.ds(16 * g, 16)
        s = par[pl.ds(3 * NPAD + 16 * g, 16)]
        cut = (3.0 * SF) * s
        inv = 0.5 / (s * s)

        def bounds(c):
            ci = c * SF
            lo = jnp.maximum(ci - cut, 0.0).astype(jnp.int32)
            hi = jnp.minimum(
                (jnp.minimum(ci + cut, SF) + 1.0).astype(jnp.int32), D)
            return lo, hi

        xlo, xhi = bounds(par[pl.ds(0 * NPAD + 16 * g, 16)])
        ylo, yhi = bounds(par[pl.ds(1 * NPAD + 16 * g, 16)])
        zlo, zhi = bounds(par[pl.ds(2 * NPAD + 16 * g, 16)])
        il = jnp.maximum(xlo - x0, 0)
        ih = jnp.minimum(xhi - x0, SLAB)
        invr[sl] = inv
        ipk[sl] = il | (ih << 8)
        ypk[sl] = ylo | (yhi << 8)
        zpk[sl] = zlo | (zhi << 8)
        cxv = par[pl.ds(0 * NPAD + 16 * g, 16)]
        for p in range(SLAB):
            xi = x0 + p
            t = xi.astype(jnp.float32) * INV_SF - cxv
            gxa[p][sl] = jnp.where((xi >= xlo) & (xi < xhi),
                                   jnp.exp(-(t * t) * inv), 0.0)
        return 0

    lax.fori_loop(0, NG, _prep, 0)

    def _gauss(n, _):
        w = _sread(ipk, n)
        il = w & 255
        ih = w >> 8

        @pl.when(il < ih)
        def _():
            _gauss_hit(n, _sread(ypk, n), _sread(zpk, n), il, ih)

        return 0

    def _gauss_hit(n, yw, zw, il, ih):
        ylo = yw & 255
        yhi = yw >> 8
        zlo = zw & 255
        zhi = zw >> 8
        inv = _sread(invr, n)
        cyn = _sread2(par, 1, n)
        czn = _sread2(par, 2, n)
        inten = _sread2(par, 4, n)
        yc0 = jnp.minimum(ylo, D - 32)
        nzc = (zhi - zlo + 15) >> 4
        zc0 = jnp.minimum(zlo, D - 16 * nzc)

        def axis_chunk(c0, ca, lo, hi, scale):
            idx = c0 + lane
            t = idx.astype(jnp.float32) * INV_SF - ca
            return jnp.where((idx >= lo) & (idx < hi),
                             jnp.exp(-(t * t) * inv) * scale, 0.0)

        gy0 = axis_chunk(yc0, cyn, ylo, yhi, inten)
        gy1 = axis_chunk(yc0 + 16, cyn, ylo, yhi, inten)
        gz0 = axis_chunk(zc0, czn, zlo, zhi, 1.0)
        gz1 = axis_chunk(zc0 + 16, czn, zlo, zhi, 1.0)

        fbuf[pl.ds(0, 16)] = gy0
        fbuf[pl.ds(16, 16)] = gy1
        a = [_sread(gxa[i], n) for i in range(SLAB)]

        @pl.when(nzc == 1)
        def _():
            @plsc.parallel_loop(ylo, yhi, unroll=4)
            def _yj(j):
                gyj = _sread(fbuf, j - yc0)
                for i in range(SLAB):
                    r = i * D + j
                    vol[r, pl.ds(zc0, 16)] = (
                        vol[r, pl.ds(zc0, 16)] + (a[i] * gyj) * gz0)

        @pl.when(nzc == 2)
        def _():
            @plsc.parallel_loop(ylo, yhi, unroll=2)
            def _yj(j):
                gyj = _sread(fbuf, j - yc0)
                for i in range(SLAB):
                    r = i * D + j
                    f = a[i] * gyj
                    vol[r, pl.ds(zc0, 16)] = (
                        vol[r, pl.ds(zc0, 16)] + f * gz0)
                    vol[r, pl.ds(zc0 + 16, 16)] = (
                        vol[r, pl.ds(zc0 + 16, 16)] + f * gz1)

    lax.fori_loop(0, N, _gauss, 0)

    pltpu.sync_copy(vol, out_h.at[pl.ds(x0 * D, SLAB * D)])


@jax.jit
def _splat(par):
    mesh = plsc.VectorSubcoreMesh(
        core_axis_name="c", subcore_axis_name="s", num_cores=2, num_subcores=16)
    f = pl.kernel(
        _splat_body,
        out_type=jax.ShapeDtypeStruct((D * D, D), jnp.float32),
        mesh=mesh,
        scratch_types=[
            pltpu.VMEM((5 * NPAD,), jnp.float32),
            pltpu.VMEM((NPAD,), jnp.int32),
            pltpu.VMEM((NPAD,), jnp.int32),
            pltpu.VMEM((NPAD,), jnp.int32),
            pltpu.VMEM((NPAD,), jnp.float32),
            pltpu.VMEM((NPAD,), jnp.float32),
            pltpu.VMEM((NPAD,), jnp.float32),
            pltpu.VMEM((NPAD,), jnp.float32),
            pltpu.VMEM((NPAD,), jnp.float32),
            pltpu.VMEM((48,), jnp.float32),
            pltpu.VMEM((SLAB * D, D), jnp.float32),
        ],
    )
    return f(par)


def kernel(centers, sigmas, intensities):
    pad = jnp.zeros((5, NPAD - N), jnp.float32)
    par = jnp.concatenate(
        [jnp.stack([centers[:, 0], centers[:, 1], centers[:, 2],
                    sigmas, intensities]), pad], axis=1).reshape(-1)
    out = _splat(par)
    return out.reshape(D, D, D)


# speedup vs baseline: 1.0080x; 1.0080x over previous
import jax
import jax.numpy as jnp
from jax import lax
from jax.experimental import pallas as pl
from jax.experimental.pallas import tpu as pltpu
from jax.experimental.pallas import tpu_sc as plsc

D = 128
N = 512
NPAD = N + 16
SF = float(D - 1)
INV_SF = 1.0 / SF
NW = 32
SLAB = D // NW
NG = N // 16


def _sread(ref, i):
    return ref[pl.ds(i, 16)][0]


def _sread2(ref, row, i):
    return ref[pl.ds(row * NPAD + i, 16)][0]


def _splat_body(par_h, out_h, par, ipk, ypk, zpk, invr, fbuf, vol):
    wid = lax.axis_index("s") * 2 + lax.axis_index("c")
    x0 = wid * SLAB
    x0f = x0.astype(jnp.float32)

    pltpu.sync_copy(par_h, par)

    zeros = jnp.zeros((16,), jnp.float32)

    @plsc.parallel_loop(0, SLAB * D, unroll=4)
    def _init(r):
        for c in range(8):
            vol[r, pl.ds(16 * c, 16)] = zeros

    lane = lax.iota(jnp.int32, 16)
    lanef = lane.astype(jnp.float32)

    def _prep(g, cnt):
        sl = pl.ds(16 * g, 16)
        s = par[pl.ds(3 * NPAD + 16 * g, 16)]
        cut = (3.0 * SF) * s
        inv = 0.5 / (s * s)

        def bounds(c):
            ci = c * SF
            lo = jnp.maximum(ci - cut, 0.0).astype(jnp.int32)
            hi = jnp.minimum(
                (jnp.minimum(ci + cut, SF) + 1.0).astype(jnp.int32), D)
            return lo, hi

        xlo, xhi = bounds(par[pl.ds(0 * NPAD + 16 * g, 16)])
        ylo, yhi = bounds(par[pl.ds(1 * NPAD + 16 * g, 16)])
        zlo, zhi = bounds(par[pl.ds(2 * NPAD + 16 * g, 16)])
        il = jnp.maximum(xlo - x0, 0)
        ih = jnp.minimum(xhi - x0, SLAB)
        invr[sl] = inv
        ipk[sl] = il | (ih << 8)
        ypk[sl] = ylo | (yhi << 8)
        zpk[sl] = zlo | (zhi << 8)
        return 0

    lax.fori_loop(0, NG, _prep, 0)

    def _gauss(n, _):
        w = _sread(ipk, n)
        il = w & 255
        ih = w >> 8

        @pl.when(il < ih)
        def _():
            _gauss_hit(n, _sread(ypk, n), _sread(zpk, n), il, ih)

        return 0

    def _gauss_hit(n, yw, zw, il, ih):
        ylo = yw & 255
        yhi = yw >> 8
        zlo = zw & 255
        zhi = zw >> 8
        inv = _sread(invr, n)
        cxn = _sread2(par, 0, n)
        cyn = _sread2(par, 1, n)
        czn = _sread2(par, 2, n)
        inten = _sread2(par, 4, n)
        yc0 = jnp.minimum(ylo, D - 32)
        nzc = (zhi - zlo + 15) >> 4
        zc0 = jnp.minimum(zlo, D - 16 * nzc)

        tx = (x0f + lanef) * INV_SF - cxn
        gx = jnp.where((lane >= il) & (lane < ih),
                       jnp.exp(-(tx * tx) * inv), 0.0)

        def axis_chunk(c0, ca, lo, hi, scale):
            idx = c0 + lane
            t = idx.astype(jnp.float32) * INV_SF - ca
            return jnp.where((idx >= lo) & (idx < hi),
                             jnp.exp(-(t * t) * inv) * scale, 0.0)

        gy0 = axis_chunk(yc0, cyn, ylo, yhi, inten)
        gy1 = axis_chunk(yc0 + 16, cyn, ylo, yhi, inten)
        gz0 = axis_chunk(zc0, czn, zlo, zhi, 1.0)
        gz1 = axis_chunk(zc0 + 16, czn, zlo, zhi, 1.0)

        fbuf[pl.ds(0, 16)] = gy0
        fbuf[pl.ds(16, 16)] = gy1
        a = [gx[i] for i in range(SLAB)]

        @pl.when(nzc == 1)
        def _():
            @plsc.parallel_loop(ylo, yhi, unroll=8)
            def _yj(j):
                gyj = _sread(fbuf, j - yc0)
                for i in range(SLAB):
                    r = i * D + j
                    vol[r, pl.ds(zc0, 16)] = (
                        vol[r, pl.ds(zc0, 16)] + (a[i] * gyj) * gz0)

        @pl.when(nzc == 2)
        def _():
            @plsc.parallel_loop(ylo, yhi, unroll=2)
            def _yj(j):
                gyj = _sread(fbuf, j - yc0)
                for i in range(SLAB):
                    r = i * D + j
                    f = a[i] * gyj
                    vol[r, pl.ds(zc0, 16)] = (
                        vol[r, pl.ds(zc0, 16)] + f * gz0)
                    vol[r, pl.ds(zc0 + 16, 16)] = (
                        vol[r, pl.ds(zc0 + 16, 16)] + f * gz1)

    lax.fori_loop(0, N, _gauss, 0)

    pltpu.sync_copy(vol, out_h.at[pl.ds(x0 * D, SLAB * D)])


@jax.jit
def _splat(par):
    mesh = plsc.VectorSubcoreMesh(
        core_axis_name="c", subcore_axis_name="s", num_cores=2, num_subcores=16)
    f = pl.kernel(
        _splat_body,
        out_type=jax.ShapeDtypeStruct((D * D, D), jnp.float32),
        mesh=mesh,
        scratch_types=[
            pltpu.VMEM((5 * NPAD,), jnp.float32),
            pltpu.VMEM((NPAD,), jnp.int32),
            pltpu.VMEM((NPAD,), jnp.int32),
            pltpu.VMEM((NPAD,), jnp.int32),
            pltpu.VMEM((NPAD,), jnp.float32),
            pltpu.VMEM((48,), jnp.float32),
            pltpu.VMEM((SLAB * D, D), jnp.float32),
        ],
    )
    return f(par)


def kernel(centers, sigmas, intensities):
    pad = jnp.zeros((5, NPAD - N), jnp.float32)
    par = jnp.concatenate(
        [jnp.stack([centers[:, 0], centers[:, 1], centers[:, 2],
                    sigmas, intensities]), pad], axis=1).reshape(-1)
    out = _splat(par)
    return out.reshape(D, D, D)


# speedup vs baseline: 1.0222x; 1.0142x over previous
import jax
import jax.numpy as jnp
from jax import lax
from jax.experimental import pallas as pl
from jax.experimental.pallas import tpu as pltpu
from jax.experimental.pallas import tpu_sc as plsc

D = 128
N = 512
NPAD = N + 16
SF = float(D - 1)
INV_SF = 1.0 / SF
NW = 32
SLAB = D // NW
NG = N // 16


def _sread(ref, i):
    return ref[pl.ds(i, 16)][0]


def _sread2(ref, row, i):
    return ref[pl.ds(row * NPAD + i, 16)][0]


def _splat_body(par_h, out_h, par, ipk, ypk, zpk, invr, fbuf, vol):
    wid = lax.axis_index("s") * 2 + lax.axis_index("c")
    x0 = wid * SLAB
    x0f = x0.astype(jnp.float32)

    pltpu.sync_copy(par_h, par)

    zeros = jnp.zeros((16,), jnp.float32)

    @plsc.parallel_loop(0, SLAB * D, unroll=4)
    def _init(r):
        for c in range(8):
            vol[r, pl.ds(16 * c, 16)] = zeros

    lane = lax.iota(jnp.int32, 16)
    lanef = lane.astype(jnp.float32)

    def _prep(g, cnt):
        sl = pl.ds(16 * g, 16)
        s = par[pl.ds(3 * NPAD + 16 * g, 16)]
        cut = (3.0 * SF) * s
        inv = 0.5 / (s * s)

        def bounds(c):
            ci = c * SF
            lo = jnp.maximum(ci - cut, 0.0).astype(jnp.int32)
            hi = jnp.minimum(
                (jnp.minimum(ci + cut, SF) + 1.0).astype(jnp.int32), D)
            return lo, hi

        xlo, xhi = bounds(par[pl.ds(0 * NPAD + 16 * g, 16)])
        ylo, yhi = bounds(par[pl.ds(1 * NPAD + 16 * g, 16)])
        zlo, zhi = bounds(par[pl.ds(2 * NPAD + 16 * g, 16)])
        il = jnp.maximum(xlo - x0, 0)
        ih = jnp.minimum(xhi - x0, SLAB)
        invr[sl] = inv
        ipk[sl] = il | (ih << 8)
        ypk[sl] = ylo | (yhi << 8)
        zpk[sl] = zlo | (zhi << 8)
        return 0

    lax.fori_loop(0, NG, _prep, 0)

    def _gauss(n, _):
        w = _sread(ipk, n)
        il = w & 255
        ih = w >> 8

        @pl.when(il < ih)
        def _():
            _gauss_hit(n, _sread(ypk, n), _sread(zpk, n), il, ih)

        return 0

    def _gauss_hit(n, yw, zw, il, ih):
        ylo = yw & 255
        yhi = yw >> 8
        zlo = zw & 255
        zhi = zw >> 8
        inv = _sread(invr, n)
        cxn = _sread2(par, 0, n)
        cyn = _sread2(par, 1, n)
        czn = _sread2(par, 2, n)
        inten = _sread2(par, 4, n)
        yc0 = jnp.minimum(ylo, D - 32)
        nzc = (zhi - zlo + 15) >> 4
        zc0 = jnp.minimum(zlo, D - 16 * nzc)

        tx = (x0f + lanef) * INV_SF - cxn
        gx = jnp.where((lane >= il) & (lane < ih),
                       jnp.exp(-(tx * tx) * inv), 0.0)

        def axis_chunk(c0, ca, lo, hi, scale):
            idx = c0 + lane
            t = idx.astype(jnp.float32) * INV_SF - ca
            return jnp.where((idx >= lo) & (idx < hi),
                             jnp.exp(-(t * t) * inv) * scale, 0.0)

        gy0 = axis_chunk(yc0, cyn, ylo, yhi, inten)
        gy1 = axis_chunk(yc0 + 16, cyn, ylo, yhi, inten)
        gz0 = axis_chunk(zc0, czn, zlo, zhi, 1.0)
        gz1 = axis_chunk(zc0 + 16, czn, zlo, zhi, 1.0)

        fbuf[pl.ds(0, 16)] = gy0
        fbuf[pl.ds(16, 16)] = gy1
        a = [gx[i] for i in range(SLAB)]

        @pl.when(nzc == 1)
        def _():
            @plsc.parallel_loop(ylo, yhi, unroll=4)
            def _yj(j):
                gyj = _sread(fbuf, j - yc0)
                for i in range(SLAB):
                    r = i * D + j
                    vol[r, pl.ds(zc0, 16)] = (
                        vol[r, pl.ds(zc0, 16)] + (a[i] * gyj) * gz0)

        @pl.when(nzc == 2)
        def _():
            @plsc.parallel_loop(ylo, yhi, unroll=2)
            def _yj(j):
                gyj = _sread(fbuf, j - yc0)
                for i in range(SLAB):
                    r = i * D + j
                    f = a[i] * gyj
                    vol[r, pl.ds(zc0, 16)] = (
                        vol[r, pl.ds(zc0, 16)] + f * gz0)
                    vol[r, pl.ds(zc0 + 16, 16)] = (
                        vol[r, pl.ds(zc0 + 16, 16)] + f * gz1)

    lax.fori_loop(0, N, _gauss, 0)

    pltpu.sync_copy(vol, out_h.at[pl.ds(x0 * D, SLAB * D)])


@jax.jit
def _splat(par):
    mesh = plsc.VectorSubcoreMesh(
        core_axis_name="c", subcore_axis_name="s", num_cores=2, num_subcores=16)
    f = pl.kernel(
        _splat_body,
        out_type=jax.ShapeDtypeStruct((D * D, D), jnp.float32),
        mesh=mesh,
        scratch_types=[
            pltpu.VMEM((5 * NPAD,), jnp.float32),
            pltpu.VMEM((NPAD,), jnp.int32),
            pltpu.VMEM((NPAD,), jnp.int32),
            pltpu.VMEM((NPAD,), jnp.int32),
            pltpu.VMEM((NPAD,), jnp.float32),
            pltpu.VMEM((48,), jnp.float32),
            pltpu.VMEM((SLAB * D, D), jnp.float32),
        ],
    )
    return f(par)


def kernel(centers, sigmas, intensities):
    pad = jnp.zeros((5, NPAD - N), jnp.float32)
    par = jnp.concatenate(
        [jnp.stack([centers[:, 0], centers[:, 1], centers[:, 2],
                    sigmas, intensities]), pad], axis=1).reshape(-1)
    out = _splat(par)
    return out.reshape(D, D, D)
